# Initial kernel scaffold; baseline (speedup 1.0000x reference)
import jax
import jax.numpy as jnp
from jax.experimental import pallas as pl


def kernel(x_stroke, x_brep, ei_intersects, ei_temp_previous, ei_represented_by, ei_brepcoplanar, ei_strokecoplanar, W_rel_head, W_self_head, W_rel_layers, W_self_layers):
    raise NotImplementedError("write your pallas kernel here")


# speedup vs baseline: 1.0010x; 1.0010x over previous
import functools
import jax
import jax.numpy as jnp
from jax import lax
from jax.experimental import pallas as pl

NS = 100000
NB = 50000
F = 32


def _matmul_kernel(x_ref, w_ref, o_ref):
    o_ref[...] = jnp.dot(x_ref[...], w_ref[...],
                         preferred_element_type=jnp.float32)


def _fused_matmul(x, w, block_rows):
    n, d = x.shape
    dout = w.shape[1]
    grid = n // block_rows
    return pl.pallas_call(
        _matmul_kernel,
        grid=(grid,),
        in_specs=[
            pl.BlockSpec((block_rows, d), lambda i: (i, 0)),
            pl.BlockSpec((d, dout), lambda i: (0, 0)),
        ],
        out_specs=pl.BlockSpec((block_rows, dout), lambda i: (i, 0)),
        out_shape=jax.ShapeDtypeStruct((n, dout), jnp.float32),
    )(x, w)


def _combine_kernel(z_ref, s0_ref, s1_ref, s4_ref, ic_ref, xp_ref, o_ref,
                    *, is_head, nrel):
    h = z_ref[..., :F]
    if nrel == 3:
        h = h + s0_ref[...] * ic_ref[...] + s1_ref[...] + s4_ref[...]
    else:
        h = h + s0_ref[...] * ic_ref[...] + s1_ref[...]
    if is_head:
        o_ref[...] = h
    else:
        o_ref[...] = jnp.maximum(h, 0.0) + xp_ref[...]


def _combine(z, s0, s1, s4, invc, xprev, is_head, block_rows):
    n = z.shape[0]
    nrel = 3 if s4 is not None else 2
    if s4 is None:
        s4 = s0
    grid = n // block_rows
    bs = lambda w: pl.BlockSpec((block_rows, w), lambda i: (i, 0))
    return pl.pallas_call(
        functools.partial(_combine_kernel, is_head=is_head, nrel=nrel),
        grid=(grid,),
        in_specs=[bs(z.shape[1]), bs(F), bs(F), bs(F), bs(1), bs(F)],
        out_specs=bs(F),
        out_shape=jax.ShapeDtypeStruct((n, F), jnp.float32),
    )(z, s0, s1, s4, invc, xprev)


def _relu_kernel(x_ref, o_ref):
    o_ref[...] = jnp.maximum(x_ref[...], 0.0)


def _relu(x, block_rows):
    n = x.shape[0]
    return pl.pallas_call(
        _relu_kernel,
        grid=(n // block_rows,),
        in_specs=[pl.BlockSpec((block_rows, F), lambda i: (i, 0))],
        out_specs=pl.BlockSpec((block_rows, F), lambda i: (i, 0)),
        out_shape=jax.ShapeDtypeStruct((n, F), jnp.float32),
    )(x)


def _seg_sum(vals, dst, n):
    return jax.ops.segment_sum(vals, dst, num_segments=n)


def _seg_max(vals, dst, n):
    m = jax.ops.segment_max(vals, dst, num_segments=n)
    return jnp.where(jnp.isfinite(m), m, 0.0)


def kernel(x_stroke, x_brep, ei_intersects, ei_temp_previous,
           ei_represented_by, ei_brepcoplanar, ei_strokecoplanar,
           W_rel_head, W_self_head, W_rel_layers, W_self_layers):
    e0s, e0d = ei_intersects[0], ei_intersects[1]
    e1s, e1d = ei_temp_previous[0], ei_temp_previous[1]
    e2s, e2d = ei_represented_by[0], ei_represented_by[1]
    e3s, e3d = ei_brepcoplanar[0], ei_brepcoplanar[1]
    e4s, e4d = ei_strokecoplanar[0], ei_strokecoplanar[1]

    ones = lambda e: jnp.ones(e.shape, jnp.float32)
    c0 = _seg_sum(ones(e0d), e0d, NS)
    c2 = _seg_sum(ones(e2d), e2d, NB)
    ic0 = (1.0 / jnp.maximum(c0, 1.0))[:, None]
    ic2 = (1.0 / jnp.maximum(c2, 1.0))[:, None]

    def w_cat_stroke(ws, wr):
        return jnp.concatenate([ws[0], wr[0], wr[1], wr[4], wr[2]], axis=1)

    def w_cat_brep(ws, wr):
        return jnp.concatenate([ws[1], wr[3]], axis=1)

    x0, x1 = x_stroke, x_brep
    for l in range(5):
        if l == 0:
            w0 = w_cat_stroke(W_self_head, W_rel_head)
            w1 = w_cat_brep(W_self_head, W_rel_head)
        else:
            w0 = w_cat_stroke(W_self_layers[l - 1], W_rel_layers[l - 1])
            w1 = w_cat_brep(W_self_layers[l - 1], W_rel_layers[l - 1])
        z0 = _fused_matmul(x0, w0, 1000)
        z1 = _fused_matmul(x1, w1, 1000)

        s0 = _seg_sum(jnp.take(z0[:, F:2 * F], e0s, axis=0), e0d, NS)
        s1 = _seg_sum(jnp.take(z0[:, 2 * F:3 * F], e1s, axis=0), e1d, NS)
        m4 = _seg_max(jnp.take(z0[:, 3 * F:4 * F], e4s, axis=0), e4d, NS)
        s2 = _seg_sum(jnp.take(z0[:, 4 * F:5 * F], e2s, axis=0), e2d, NB)
        m3 = _seg_max(jnp.take(z1[:, F:2 * F], e3s, axis=0), e3d, NB)

        is_head = (l == 0)
        x0 = _combine(z0, s0, s1, m4, ic0, x0 if not is_head else z0[:, :F],
                      is_head, 1000)
        x1 = _combine(z1, s2, m3, None, ic2,
                      x1 if not is_head else z1[:, :F], is_head, 1000)

    return (_relu(x0, 1000), _relu(x1, 1000))


# speedup vs baseline: 2.8035x; 2.8008x over previous
import functools
import jax
import jax.numpy as jnp
from jax import lax
from jax.experimental import pallas as pl
from jax.experimental.pallas import tpu as pltpu
from jax.experimental.pallas import tpu_sc as plsc

NS = 100000
NB = 50000
F = 32
C = 400
ZC = 256
NDUMP = 16
ACC = 50176

E0, E1, E2 = 1600000, 100000, 200000


def _matmul_kernel(x_ref, w_ref, *o_refs):
    z = jnp.dot(x_ref[...], w_ref[...], preferred_element_type=jnp.float32)
    for i, o in enumerate(o_refs):
        o[...] = z[:, i * F:(i + 1) * F]


def _fused_matmul(x, w, nout, block_rows):
    n, d = x.shape
    grid = n // block_rows
    obs = pl.BlockSpec((block_rows, F), lambda i: (i, 0))
    return pl.pallas_call(
        _matmul_kernel,
        grid=(grid,),
        in_specs=[
            pl.BlockSpec((block_rows, d), lambda i: (i, 0)),
            pl.BlockSpec((d, nout * F), lambda i: (0, 0)),
        ],
        out_specs=[obs] * nout,
        out_shape=[jax.ShapeDtypeStruct((n, F), jnp.float32)] * nout,
    )(x, w)


def _combine_kernel(z_ref, s0_ref, s1_ref, s4_ref, c_ref, xp_ref, o_ref,
                    *, is_head, nrel):
    ic = 1.0 / jnp.maximum(c_ref[...], 1.0)
    h = z_ref[...] + s0_ref[...] * ic + s1_ref[...]
    if nrel == 3:
        h = h + s4_ref[...]
    if is_head:
        o_ref[...] = h
    else:
        o_ref[...] = jnp.maximum(h, 0.0) + xp_ref[...]


def _combine(z, s0, s1, s4, cnt, xprev, is_head, block_rows):
    n = z.shape[0]
    nrel = 3 if s4 is not None else 2
    if s4 is None:
        s4 = s0
    grid = n // block_rows
    bs = lambda w: pl.BlockSpec((block_rows, w), lambda i: (i, 0))
    return pl.pallas_call(
        functools.partial(_combine_kernel, is_head=is_head, nrel=nrel),
        grid=(grid,),
        in_specs=[bs(F), bs(F), bs(F), bs(F), bs(1), bs(F)],
        out_specs=bs(F),
        out_shape=jax.ShapeDtypeStruct((n, F), jnp.float32),
    )(z, s0, s1, s4, cnt, xprev)


def _relu_kernel(x_ref, o_ref):
    o_ref[...] = jnp.maximum(x_ref[...], 0.0)


def _relu(x, block_rows):
    n = x.shape[0]
    return pl.pallas_call(
        _relu_kernel,
        grid=(n // block_rows,),
        in_specs=[pl.BlockSpec((block_rows, F), lambda i: (i, 0))],
        out_specs=pl.BlockSpec((block_rows, F), lambda i: (i, 0)),
        out_shape=jax.ShapeDtypeStruct((n, F), jnp.float32),
    )(x)


def _mesh():
    return plsc.VectorSubcoreMesh(core_axis_name="c", subcore_axis_name="s")


def _zero_zbuf(zbuf, nrows):
    def zb(i, carry):
        r = i // 2
        cs = (i % 2) * 16
        zbuf[r, pl.ds(cs, 16)] = jnp.zeros((16,), jnp.float32)
        return carry
    lax.fori_loop(0, nrows * 2, zb, 0)


def _sc_sums(zr0, zr1, zr2, s0a, d0a, s1a, d1a, s2a, d2a):
    @functools.partial(
        pl.kernel, mesh=_mesh(),
        compiler_params=pltpu.CompilerParams(use_tc_tiling_on_sc=False),
        out_type=[jax.ShapeDtypeStruct((NS, F), jnp.float32),
                  jax.ShapeDtypeStruct((NS, F), jnp.float32),
                  jax.ShapeDtypeStruct((NB, F), jnp.float32)],
        scratch_types=[
            pltpu.VMEM((C,), jnp.int32),
            pltpu.VMEM((C,), jnp.int32),
            pltpu.VMEM((C,), jnp.int32),
            pltpu.VMEM((C, F), jnp.float32),
            pltpu.VMEM((ZC, F), jnp.float32),
            pltpu.VMEM_SHARED((ACC, F), jnp.float32),
            pltpu.SemaphoreType.DMA,
        ],
    )
    def k(z0h, z1h, z2h, s0h, d0h, s1h, d1h, s2h, d2h,
          o0h, o1h, o2h, src_v, dst_v, idx_v, rows_v, zbuf, acc, sem):
        cc = lax.axis_index("c")
        ss = lax.axis_index("s")
        iot = lax.iota(jnp.int32, 16)
        _zero_zbuf(zbuf, ZC)

        for (zh, sh, dh, oh, E, ndst) in ((z0h, s0h, d0h, o0h, E0, NS),
                                          (z1h, s1h, d1h, o1h, E1, NS),
                                          (z2h, s2h, d2h, o2h, E2, NB)):
            half = ndst // 2
            base = cc * half
            nzc = (half + NDUMP + ZC - 1) // ZC

            def zr(t, carry):
                kk = t * 16 + ss

                @pl.when(kk < nzc)
                def _():
                    pltpu.sync_copy(zbuf, acc.at[pl.ds(kk * ZC, ZC)])
                return carry
            lax.fori_loop(0, (nzc + 15) // 16, zr, 0)
            plsc.subcore_barrier()

            nch = E // C

            def chunk(t, carry):
                j = t * 16 + ss

                @pl.when(j < nch)
                def _():
                    off = j * C
                    pltpu.sync_copy(sh.at[pl.ds(off, C)], src_v)
                    pltpu.sync_copy(dh.at[pl.ds(off, C)], dst_v)

                    def ib(i, carry2):
                        d = dst_v[pl.ds(i * 16, 16)]
                        li = d - base
                        oob = (li < 0) | (li >= half)
                        li = jnp.where(oob, half + iot, li)
                        idx_v[pl.ds(i * 16, 16)] = li
                        return carry2
                    lax.fori_loop(0, C // 16, ib, 0)
                    pltpu.async_copy(zh.at[src_v], rows_v, sem).wait()
                    pltpu.sync_copy(rows_v, acc.at[idx_v], add=True)
                return carry
            lax.fori_loop(0, (nch + 15) // 16, chunk, 0)
            plsc.subcore_barrier()

            @pl.when(ss == 0)
            def _():
                pltpu.sync_copy(acc.at[pl.ds(0, half)],
                                oh.at[pl.ds(base, half)])
            plsc.subcore_barrier()

    return k(zr0, zr1, zr2, s0a, d0a, s1a, d1a, s2a, d2a)


def _sc_counts(d0a, d2a):
    @functools.partial(
        pl.kernel, mesh=_mesh(),
        compiler_params=pltpu.CompilerParams(use_tc_tiling_on_sc=False),
        out_type=[jax.ShapeDtypeStruct((NS,), jnp.float32),
                  jax.ShapeDtypeStruct((NB,), jnp.float32)],
        scratch_types=[
            pltpu.VMEM((C,), jnp.int32),
            pltpu.VMEM((C,), jnp.int32),
            pltpu.VMEM((C,), jnp.float32),
            pltpu.VMEM((ZC,), jnp.float32),
            pltpu.VMEM_SHARED((ACC,), jnp.float32),
        ],
    )
    def k(d0h, d2h, o0h, o2h, dst_v, idx_v, ones_v, zbuf, acc):
        cc = lax.axis_index("c")
        ss = lax.axis_index("s")
        iot = lax.iota(jnp.int32, 16)

        def fill(i, carry):
            zbuf[pl.ds(i * 16, 16)] = jnp.zeros((16,), jnp.float32)
            return carry
        lax.fori_loop(0, ZC // 16, fill, 0)

        def fill1(i, carry):
            ones_v[pl.ds(i * 16, 16)] = jnp.full((16,), 1.0, jnp.float32)
            return carry
        lax.fori_loop(0, C // 16, fill1, 0)

        for (dh, oh, E, ndst) in ((d0h, o0h, E0, NS), (d2h, o2h, E2, NB)):
            half = ndst // 2
            base = cc * half
            nzc = (half + NDUMP + ZC - 1) // ZC

            def zr(t, carry):
                kk = t * 16 + ss

                @pl.when(kk < nzc)
                def _():
                    pltpu.sync_copy(zbuf, acc.at[pl.ds(kk * ZC, ZC)])
                return carry
            lax.fori_loop(0, (nzc + 15) // 16, zr, 0)
            plsc.subcore_barrier()

            nch = E // C

            def chunk(t, carry):
                j = t * 16 + ss

                @pl.when(j < nch)
                def _():
                    pltpu.sync_copy(dh.at[pl.ds(j * C, C)], dst_v)

                    def ib(i, carry2):
                        d = dst_v[pl.ds(i * 16, 16)]
                        li = d - base
                        oob = (li < 0) | (li >= half)
                        li = jnp.where(oob, half + iot, li)
                        idx_v[pl.ds(i * 16, 16)] = li
                        return carry2
                    lax.fori_loop(0, C // 16, ib, 0)
                    pltpu.sync_copy(ones_v, acc.at[idx_v], add=True)
                return carry
            lax.fori_loop(0, (nch + 15) // 16, chunk, 0)
            plsc.subcore_barrier()

            @pl.when(ss == 0)
            def _():
                pltpu.sync_copy(acc.at[pl.ds(0, half)],
                                oh.at[pl.ds(base, half)])
            plsc.subcore_barrier()

    return k(d0a, d2a)


def _seg_max(vals, dst, n):
    m = jax.ops.segment_max(vals, dst, num_segments=n)
    return jnp.where(jnp.isfinite(m), m, 0.0)


def kernel(x_stroke, x_brep, ei_intersects, ei_temp_previous,
           ei_represented_by, ei_brepcoplanar, ei_strokecoplanar,
           W_rel_head, W_self_head, W_rel_layers, W_self_layers):
    e0s, e0d = ei_intersects[0], ei_intersects[1]
    e1s, e1d = ei_temp_previous[0], ei_temp_previous[1]
    e2s, e2d = ei_represented_by[0], ei_represented_by[1]
    e3s, e3d = ei_brepcoplanar[0], ei_brepcoplanar[1]
    e4s, e4d = ei_strokecoplanar[0], ei_strokecoplanar[1]

    c0w, c2w = _sc_counts(e0d, e2d)
    c0 = c0w[:, None]
    c2 = c2w[:, None]

    def w_cat_stroke(ws, wr):
        return jnp.concatenate([ws[0], wr[0], wr[1], wr[4], wr[2]], axis=1)

    def w_cat_brep(ws, wr):
        return jnp.concatenate([ws[1], wr[3]], axis=1)

    x0, x1 = x_stroke, x_brep
    for l in range(5):
        if l == 0:
            w0 = w_cat_stroke(W_self_head, W_rel_head)
            w1 = w_cat_brep(W_self_head, W_rel_head)
        else:
            w0 = w_cat_stroke(W_self_layers[l - 1], W_rel_layers[l - 1])
            w1 = w_cat_brep(W_self_layers[l - 1], W_rel_layers[l - 1])
        zs0, zr0, zr1, zr4, zr2 = _fused_matmul(x0, w0, 5, 1000)
        zs1, zr3 = _fused_matmul(x1, w1, 2, 1000)

        s0, s1, s2 = _sc_sums(zr0, zr1, zr2, e0s, e0d, e1s, e1d, e2s, e2d)
        m4 = _seg_max(jnp.take(zr4, e4s, axis=0), e4d, NS)
        m3 = _seg_max(jnp.take(zr3, e3s, axis=0), e3d, NB)

        is_head = (l == 0)
        x0 = _combine(zs0, s0, s1, m4, c0, x0 if not is_head else zs0,
                      is_head, 1000)
        x1 = _combine(zs1, s2, m3, None, c2, x1 if not is_head else zs1,
                      is_head, 1000)

    return (_relu(x0, 1000), _relu(x1, 1000))
